# Initial kernel scaffold; baseline (speedup 1.0000x reference)
import jax
import jax.numpy as jnp
from jax.experimental import pallas as pl


def kernel(seq1, seq2, adj, weight, msk, samp_bias1, samp_bias2, W1, b1, W2, b2, W_bil, b_bil):
    raise NotImplementedError("write your pallas kernel here")


# speedup vs baseline: 11.6988x; 11.6988x over previous
import functools

import jax
import jax.numpy as jnp
from jax import lax
from jax.experimental import pallas as pl
from jax.experimental.pallas import tpu as pltpu
from jax.experimental.pallas import tpu_sc as plsc

_N = 10000
_NP = 10240
_NH = _NP // 2
_NT = _NH + 128
_E = 320000
_CH = 128
_EP = 327680
_NSUB = 16
_NCH = _EP // _NSUB // _CH
_SCH = 16
_NSC = _NCH // _SCH
_RPA = _NT // _NSUB
_RPF = _NH // _NSUB

_mesh = plsc.VectorSubcoreMesh(core_axis_name="c", subcore_axis_name="s")


def _bcast_lane(wv, l):
    return wv.at[jnp.full((16,), l, jnp.int32)].get(mode="promise_in_bounds")


def _zero_rows(buf, nrows):
    def z(i, carry):
        r = i // 8
        j = i % 8
        buf[r, pl.ds(j * 16, 16)] = jnp.zeros((16,), jnp.float32)
        return carry
    lax.fori_loop(0, nrows * 8, z, 0)


def _zero_acc(buf, acc, row0):
    for k0, sz in ((0, 128), (128, 128), (256, _RPA - 256)):
        pltpu.sync_copy(buf.at[pl.ds(0, sz)],
                        acc.at[pl.ds(row0 + k0, sz)])


def _flush_acc(acc, out_at, row0):
    for k0, sz in ((0, 128), (128, 128), (256, _RPF - 256)):
        pltpu.sync_copy(acc.at[pl.ds(row0 + k0, sz)],
                        out_at.at[pl.ds(row0 + k0, sz)])


def _clamp_idx(dst_v, slot, jg, base, idx_v):
    def cl(j, carry):
        sl16 = pl.ds(j * 16, 16)
        dv = dst_v[slot, jg, sl16]
        loc = dv - base
        inb = jnp.logical_and(loc >= 0, loc < _NH)
        trash = _NH + jnp.bitwise_and(dv, 127)
        idx_v[0, sl16] = jnp.where(inb, loc, trash)
        return carry
    lax.fori_loop(0, _CH // 16, cl, 0)


@functools.partial(
    pl.kernel, mesh=_mesh,
    out_type=jax.ShapeDtypeStruct((2, _NH, 128), jnp.float32),
    scratch_types=[
        pltpu.VMEM((_NCH, _CH), jnp.int32),
        pltpu.VMEM((_NCH, _CH), jnp.float32),
        pltpu.VMEM((_CH, 128), jnp.float32),
        pltpu.VMEM((1, _CH), jnp.int32),
        pltpu.VMEM_SHARED((_NT, 128), jnp.float32),
    ],
)
def _sc_degree(dst_hbm, w_hbm, out_hbm, dst_v, w_v, buf, idx_v, acc):
    c = lax.axis_index("c")
    s = lax.axis_index("s")
    pltpu.sync_copy(dst_hbm.at[s], dst_v)
    pltpu.sync_copy(w_hbm.at[s], w_v)

    base = c * _NH
    _zero_rows(buf, _CH)
    _zero_acc(buf, acc, s * _RPA)
    plsc.subcore_barrier()

    lane0 = lax.iota(jnp.int32, 16) == 0

    def chunk(g, carry):
        def fill16(q, carry2):
            e0 = q * 16
            wv = w_v[g, pl.ds(e0, 16)]
            for l in range(16):
                buf[e0 + l, pl.ds(0, 16)] = jnp.where(
                    lane0, _bcast_lane(wv, l), 0.0)
            return carry2
        lax.fori_loop(0, _CH // 16, fill16, 0)

        def cl(j, carry2):
            sl16 = pl.ds(j * 16, 16)
            dv = dst_v[g, sl16]
            loc = dv - base
            inb = jnp.logical_and(loc >= 0, loc < _NH)
            trash = _NH + jnp.bitwise_and(dv, 127)
            idx_v[0, sl16] = jnp.where(inb, loc, trash)
            return carry2
        lax.fori_loop(0, _CH // 16, cl, 0)
        pltpu.sync_copy(buf, acc.at[idx_v.at[0]], add=True)
        return carry
    lax.fori_loop(0, _NCH, chunk, 0)

    plsc.subcore_barrier()
    _flush_acc(acc, out_hbm.at[c], s * _RPF)


def _dinv_body(p_ref, o_ref):
    deg = p_ref[...] + 1.0
    o_ref[...] = jnp.where(deg > 0, lax.rsqrt(deg), 0.0)


def _tc_dinv(pcol):
    return pl.pallas_call(
        _dinv_body,
        out_shape=jax.ShapeDtypeStruct((_NP, 1), jnp.float32),
    )(pcol)


def _mm_body(x_ref, w_ref, d_ref, o0_ref, o1_ref):
    xw = jnp.dot(x_ref[0], w_ref[...], preferred_element_type=jnp.float32)
    xw = xw * d_ref[...]
    o0_ref[0] = xw[:, 0:128]
    o1_ref[0] = xw[:, 128:256]


def _tc_xw(x, W1, dinv):
    blk = 1024
    grid = (2, _NP // blk)
    hspec = pl.BlockSpec((1, blk, 128), lambda s, i: (s, i, 0))
    hshape = jax.ShapeDtypeStruct((2, _NP, 128), jnp.float32)
    return pl.pallas_call(
        _mm_body,
        grid=grid,
        in_specs=[
            pl.BlockSpec((1, blk, 128), lambda s, i: (s, i, 0)),
            pl.BlockSpec((128, 256), lambda s, i: (0, 0)),
            pl.BlockSpec((blk, 1), lambda s, i: (i, 0)),
        ],
        out_specs=[hspec, hspec],
        out_shape=[hshape, hshape],
    )(x, W1, dinv)


@functools.partial(
    pl.kernel, mesh=_mesh,
    out_type=jax.ShapeDtypeStruct((2, 2, 2, _NH, 128), jnp.float32),
    scratch_types=[
        pltpu.VMEM((2, _SCH, _CH), jnp.int32),
        pltpu.VMEM((2, _SCH, _CH), jnp.int32),
        pltpu.VMEM((2, _SCH, _CH), jnp.float32),
        pltpu.VMEM((2, _CH, 128), jnp.float32),
        pltpu.VMEM((1, _CH), jnp.int32),
        pltpu.VMEM_SHARED((_NT, 128), jnp.float32),
        pltpu.SemaphoreType.DMA,
        pltpu.SemaphoreType.DMA,
        pltpu.SemaphoreType.DMA,
        pltpu.SemaphoreType.DMA,
    ],
)
def _sc_spmm(tab_hbm, src_hbm, dst_hbm, w_hbm, out_hbm,
             src_v, dst_v, w_v, buf, idx_v, acc, sem0, sem1, st0, st1):
    c = lax.axis_index("c")
    s = lax.axis_index("s")
    sems = (sem0, sem1)
    stsems = (st0, st1)

    def stage_start(sc, slot):
        stsem = stsems[slot]
        sl = pl.ds(sc * _SCH, _SCH)
        pltpu.async_copy(src_hbm.at[s, sl], src_v.at[slot], stsem)
        pltpu.async_copy(dst_hbm.at[s, sl], dst_v.at[slot], stsem)
        pltpu.async_copy(w_hbm.at[s, sl], w_v.at[slot], stsem)

    def stage_wait_bias(sc, slot, bias):
        stsem = stsems[slot]
        sl = pl.ds(sc * _SCH, _SCH)
        pltpu.make_async_copy(src_hbm.at[s, sl], src_v.at[slot], stsem).wait()
        pltpu.make_async_copy(dst_hbm.at[s, sl], dst_v.at[slot], stsem).wait()
        pltpu.make_async_copy(w_hbm.at[s, sl], w_v.at[slot], stsem).wait()

        def add_bias(i, carry):
            r = i // 8
            j = i % 8
            sl16 = pl.ds(j * 16, 16)
            src_v[slot, r, sl16] = src_v[slot, r, sl16] + bias
            return carry
        lax.fori_loop(0, _SCH * 8, add_bias, 0)

    def do_chunk(b, slot, jg, base):
        pltpu.make_async_copy(tab_hbm.at[src_v.at[slot, jg]],
                              buf.at[b], sems[b]).wait()

        def scale16(i16, carry2):
            e0 = i16 * 16
            wv = w_v[slot, jg, pl.ds(e0, 16)]
            for l in range(16):
                e = e0 + l
                wb = _bcast_lane(wv, l)
                for jj in range(8):
                    sl = pl.ds(jj * 16, 16)
                    buf[b, e, sl] = buf[b, e, sl] * wb
            return carry2
        lax.fori_loop(0, _CH // 16, scale16, 0)
        _clamp_idx(dst_v, slot, jg, base, idx_v)
        pltpu.sync_copy(buf.at[b], acc.at[idx_v.at[0]], add=True)

    def phase(ph, carry):
        sq = ph // 2
        p = ph % 2
        bias = (sq * 2 + c) * _NP
        base = p * _NH

        _zero_rows(buf.at[0], _CH)
        _zero_acc(buf.at[0], acc, s * _RPA)
        plsc.subcore_barrier()

        stage_start(0, 0)
        stage_start(1, 1)
        stage_wait_bias(0, 0, bias)
        for b in range(2):
            pltpu.async_copy(tab_hbm.at[src_v.at[0, b]], buf.at[b], sems[b])

        def section(sc, sslot):
            def pair(cc, c2):
                for b in range(2):
                    jg = cc * 2 + b
                    do_chunk(b, sslot, jg, base)
                    pltpu.async_copy(tab_hbm.at[src_v.at[sslot, jg + 2]],
                                     buf.at[b], sems[b])
                return c2
            lax.fori_loop(0, _SCH // 2 - 1, pair, 0)

            @pl.when(sc + 1 < _NSC)
            def _wait_next():
                stage_wait_bias(sc + 1, 1 - sslot, bias)
            for b in range(2):
                do_chunk(b, sslot, _SCH - 2 + b, base)

                @pl.when(sc + 1 < _NSC)
                def _issue_next():
                    pltpu.async_copy(tab_hbm.at[src_v.at[1 - sslot, b]],
                                     buf.at[b], sems[b])

            @pl.when(sc + 2 < _NSC)
            def _prefetch():
                stage_start(sc + 2, sslot)

        def superpair(sp, c2):
            section(sp * 2, 0)
            section(sp * 2 + 1, 1)
            return c2
        lax.fori_loop(0, _NSC // 2, superpair, 0)

        plsc.subcore_barrier()
        _flush_acc(acc, out_hbm.at[sq, c, p], s * _RPF)
        plsc.subcore_barrier()
        return carry
    lax.fori_loop(0, 4, phase, 0)


def _mid_body(acc_ref, xs_ref, d_ref, w2_ref, wbil_ref, b1_ref, o_ref):
    d = d_ref[...]
    pre = acc_ref[...] + xs_ref[...]
    h = jnp.concatenate([pre[:, 0], pre[:, 1]], axis=-1)
    r = jax.nn.relu(h * d[None] + b1_ref[...][None])
    v = jnp.sum(wbil_ref[0], axis=1, keepdims=True)
    w2v = jnp.dot(w2_ref[...], v, preferred_element_type=jnp.float32)
    w2o = jnp.sum(w2_ref[...], axis=1, keepdims=True)
    y0 = jnp.dot(r[0], w2v, preferred_element_type=jnp.float32)
    y1 = jnp.dot(r[0], w2o, preferred_element_type=jnp.float32)
    y2 = jnp.dot(r[1], w2v, preferred_element_type=jnp.float32)
    blk = y0.shape[0]
    y = jnp.concatenate(
        [y0, y1, y2, jnp.zeros((blk, 125), jnp.float32)], axis=1)
    o_ref[...] = y * d


def _tc_mid(acc, xs, dinv, W2, W_bil, b1):
    blk = 1024
    grid = (_NP // blk,)
    return pl.pallas_call(
        _mid_body,
        grid=grid,
        in_specs=[
            pl.BlockSpec((2, 2, blk, 128), lambda i: (0, 0, i, 0)),
            pl.BlockSpec((2, 2, blk, 128), lambda i: (0, 0, i, 0)),
            pl.BlockSpec((blk, 1), lambda i: (i, 0)),
            pl.BlockSpec((256, 256), lambda i: (0, 0)),
            pl.BlockSpec((1, 256, 256), lambda i: (0, 0, 0)),
            pl.BlockSpec((1, 256), lambda i: (0, 0)),
        ],
        out_specs=pl.BlockSpec((blk, 128), lambda i: (i, 0)),
        out_shape=jax.ShapeDtypeStruct((_NP, 128), jnp.float32),
    )(acc, xs, dinv, W2, W_bil, b1)


@functools.partial(
    pl.kernel, mesh=_mesh,
    out_type=jax.ShapeDtypeStruct((2, _NH, 128), jnp.float32),
    scratch_types=[
        pltpu.VMEM((2, _SCH, _CH), jnp.int32),
        pltpu.VMEM((2, _SCH, _CH), jnp.int32),
        pltpu.VMEM((2, _SCH, _CH), jnp.float32),
        pltpu.VMEM((2, _CH, 128), jnp.float32),
        pltpu.VMEM((1, _CH), jnp.int32),
        pltpu.VMEM_SHARED((_NT, 128), jnp.float32),
        pltpu.SemaphoreType.DMA,
        pltpu.SemaphoreType.DMA,
        pltpu.SemaphoreType.DMA,
        pltpu.SemaphoreType.DMA,
    ],
)
def _sc_spmv(tab_hbm, src_hbm, dst_hbm, w_hbm, out_hbm,
             src_v, dst_v, w_v, buf, idx_v, acc, sem0, sem1, st0, st1):
    c = lax.axis_index("c")
    s = lax.axis_index("s")
    base = c * _NH
    sems = (sem0, sem1)
    stsems = (st0, st1)

    def stage_start(sc, slot):
        stsem = stsems[slot]
        sl = pl.ds(sc * _SCH, _SCH)
        pltpu.async_copy(src_hbm.at[s, sl], src_v.at[slot], stsem)
        pltpu.async_copy(dst_hbm.at[s, sl], dst_v.at[slot], stsem)
        pltpu.async_copy(w_hbm.at[s, sl], w_v.at[slot], stsem)

    def stage_wait(sc, slot):
        stsem = stsems[slot]
        sl = pl.ds(sc * _SCH, _SCH)
        pltpu.make_async_copy(src_hbm.at[s, sl], src_v.at[slot],
                              stsem).wait()
        pltpu.make_async_copy(dst_hbm.at[s, sl], dst_v.at[slot],
                              stsem).wait()
        pltpu.make_async_copy(w_hbm.at[s, sl], w_v.at[slot], stsem).wait()

    _zero_rows(buf.at[0], _CH)
    _zero_acc(buf.at[0], acc, s * _RPA)
    plsc.subcore_barrier()

    stage_start(0, 0)
    stage_start(1, 1)
    stage_wait(0, 0)
    for b in range(2):
        pltpu.async_copy(tab_hbm.at[src_v.at[0, b]], buf.at[b], sems[b])

    def do_chunk(b, slot, jg):
        pltpu.make_async_copy(tab_hbm.at[src_v.at[slot, jg]],
                              buf.at[b], sems[b]).wait()

        def scale16(i16, carry2):
            e0 = i16 * 16
            wv = w_v[slot, jg, pl.ds(e0, 16)]
            for l in range(16):
                e = e0 + l
                wb = _bcast_lane(wv, l)
                buf[b, e, pl.ds(0, 16)] = buf[b, e, pl.ds(0, 16)] * wb
            return carry2
        lax.fori_loop(0, _CH // 16, scale16, 0)
        _clamp_idx(dst_v, slot, jg, base, idx_v)
        pltpu.sync_copy(buf.at[b], acc.at[idx_v.at[0]], add=True)

    def section(sc, sslot):
        def pair(cc, c2):
            for b in range(2):
                jg = cc * 2 + b
                do_chunk(b, sslot, jg)
                pltpu.async_copy(tab_hbm.at[src_v.at[sslot, jg + 2]],
                                 buf.at[b], sems[b])
            return c2
        lax.fori_loop(0, _SCH // 2 - 1, pair, 0)

        @pl.when(sc + 1 < _NSC)
        def _wait_next():
            stage_wait(sc + 1, 1 - sslot)
        for b in range(2):
            do_chunk(b, sslot, _SCH - 2 + b)

            @pl.when(sc + 1 < _NSC)
            def _issue_next():
                pltpu.async_copy(tab_hbm.at[src_v.at[1 - sslot, b]],
                                 buf.at[b], sems[b])

        @pl.when(sc + 2 < _NSC)
        def _prefetch():
            stage_start(sc + 2, sslot)

    def superpair(sp, c2):
        section(sp * 2, 0)
        section(sp * 2 + 1, 1)
        return c2
    lax.fori_loop(0, _NSC // 2, superpair, 0)

    plsc.subcore_barrier()
    _flush_acc(acc, out_hbm.at[c], s * _RPF)


def _fin_body(a3_ref, ysc_ref, d_ref, msk_ref, sb1_ref, sb2_ref,
              wbil_ref, b2_ref, bbil_ref, o_ref):
    d = d_ref[...]
    z0 = (a3_ref[0] + ysc_ref[0]) * d
    z1 = (a3_ref[1] + ysc_ref[1]) * d
    z2 = (a3_ref[2] + ysc_ref[2]) * d
    v = jnp.sum(wbil_ref[0], axis=1, keepdims=True)
    b2v = jnp.sum(jnp.dot(b2_ref[...], v,
                          preferred_element_type=jnp.float32))
    b2s = jnp.sum(b2_ref[...])
    h1v = z0 + b2v
    h1s = z1 + b2s
    h2v = z2 + b2v
    msk = msk_ref[...]
    cvec = jax.nn.sigmoid(msk * h1s / jnp.sum(msk))
    bb = bbil_ref[0, 0]
    o_ref[0] = cvec * h1v + bb + sb1_ref[...]
    o_ref[1] = cvec * h2v + bb + sb2_ref[...]


def _tc_fin(a3, ysc, dinv2, msk2, sb12, sb22, W_bil, b2, b_bil):
    return pl.pallas_call(
        _fin_body,
        out_shape=jax.ShapeDtypeStruct((2, _NP // 128, 128), jnp.float32),
    )(a3, ysc, dinv2, msk2, sb12, sb22, W_bil, b2, b_bil)


def kernel(seq1, seq2, adj, weight, msk, samp_bias1, samp_bias2,
           W1, b1, W2, b2, W_bil, b_bil):
    f32 = jnp.float32
    src = adj[0]
    dst = adj[1]
    npad_e = _EP - _E
    pad_idx = _N + (jnp.arange(npad_e, dtype=jnp.int32) % (_NP - _N))
    src_p = jnp.concatenate([src, pad_idx])
    dst_p = jnp.concatenate([dst, pad_idx])
    w_p = jnp.concatenate([weight, jnp.zeros((npad_e,), f32)])

    src16 = src_p.reshape(_NSUB, _NCH, _CH)
    dst16 = dst_p.reshape(_NSUB, _NCH, _CH)
    w16 = w_p.reshape(_NSUB, _NCH, _CH)

    partials = _sc_degree(dst16, w16)
    dinv = _tc_dinv(partials[:, :, 0].reshape(_NP, 1))

    x = jnp.stack([seq1, seq2])
    x = jnp.pad(x, ((0, 0), (0, _NP - _N), (0, 0)))
    halves = _tc_xw(x, W1, dinv)
    xs = jnp.stack(halves, axis=1)

    tab = xs.reshape(4 * _NP, 128)
    accr = _sc_spmm(tab, src16, dst16, w16)
    acc = accr.reshape(2, 2, _NP, 128)

    ys = _tc_mid(acc, xs, dinv, W2, W_bil, b1.reshape(1, 256))

    acc3 = _sc_spmv(ys, src16, dst16, w16).reshape(_NP, 128)

    rows2 = (_NP // 128, 128)
    mskp = jnp.pad(msk, (0, _NP - _N)).reshape(rows2)
    sb1p = jnp.pad(samp_bias1, (0, _NP - _N)).reshape(rows2)
    sb2p = jnp.pad(samp_bias2, (0, _NP - _N)).reshape(rows2)
    a3 = jnp.transpose(acc3[:, :3], (1, 0)).reshape((3,) + rows2)
    ysc = jnp.transpose(ys[:, :3], (1, 0)).reshape((3,) + rows2)
    dinv2 = dinv.reshape(rows2)
    out = _tc_fin(a3, ysc, dinv2, mskp, sb1p, sb2p,
                  W_bil, b2.reshape(1, 256), b_bil.reshape(1, 1))
    flat = out.reshape(2, _NP)
    return jnp.concatenate([flat[0, :_N], flat[1, :_N]])
